# baseline (speedup 1.0000x reference)
import functools

import jax
import jax.numpy as jnp
from jax import lax
from jax.experimental import pallas as pl
from jax.experimental.pallas import tpu as pltpu
from jax.experimental.pallas import tpu_sc as plsc

N = 32768
L = 16
ROWS = N // L
WIDE = 128
WROWS = N // WIDE
NS = 16
R_TILE = ROWS // NS
W_TILE = WROWS // NS
R_UNROLL = 8
SH_PAD = 32
NEG_INF = float("-inf")

_mesh = plsc.VectorSubcoreMesh(
    core_axis_name="c", subcore_axis_name="s", num_cores=1
)


@functools.partial(
    pl.kernel,
    mesh=_mesh,
    compiler_params=pltpu.CompilerParams(needs_layout_passes=False),
    out_type=jax.ShapeDtypeStruct((WROWS, WIDE), jnp.float32),
    scratch_types=[
        pltpu.VMEM((R_TILE, L), jnp.float32),
        pltpu.VMEM((W_TILE, WIDE), jnp.float32),
        pltpu.VMEM((2, L), jnp.float32),
        pltpu.VMEM((L,), jnp.float32),
        pltpu.VMEM((L,), jnp.int32),
        pltpu.VMEM((2 * NS, L), jnp.float32),
        pltpu.SemaphoreType.DMA,
        pltpu.SemaphoreType.DMA,
        pltpu.VMEM_SHARED((SH_PAD + 2 * NS, L), jnp.float32),
    ],
)
def _argmax_onehot(x_hbm, out_hbm, xv, ov, stage, tm, ti, lms,
                   sem_in, sem_out, sh):
    sid = lax.axis_index("s")
    row0 = sid * R_TILE
    wrow0 = sid * W_TILE

    in_dma = pltpu.async_copy(x_hbm.at[pl.ds(row0, R_TILE), :], xv, sem_in)

    zero = jnp.zeros((L,), jnp.float32)
    for r in range(W_TILE):
        for c in range(WIDE // L):
            ov[r, pl.ds(c * L, L)] = zero
    out_dma = pltpu.async_copy(ov, out_hbm.at[pl.ds(wrow0, W_TILE), :], sem_out)

    in_dma.wait()
    lane = lax.broadcasted_iota(jnp.int32, (L,), 0)

    def rbody(j, carry):
        mv, iv, cur = carry
        base = j * R_UNROLL
        for k in range(R_UNROLL):
            v = xv[base + k]
            idx = cur + k * L
            better = v > mv
            mv = jnp.where(better, v, mv)
            iv = jnp.where(better, idx, iv)
        return mv, iv, cur + R_UNROLL * L

    mv0 = jnp.full((L,), NEG_INF, jnp.float32)
    iv0 = jnp.zeros((L,), jnp.int32)
    cur0 = lane + row0 * L
    mv, iv, _ = lax.fori_loop(0, R_TILE // R_UNROLL, rbody, (mv0, iv0, cur0))

    stage[0] = mv
    stage[1] = plsc.bitcast(iv, jnp.float32)
    pltpu.sync_copy(stage, sh.at[pl.ds(SH_PAD + 2 * sid, 2)])
    plsc.subcore_barrier()

    pltpu.sync_copy(sh.at[pl.ds(SH_PAD, 2 * NS)], lms)

    gm = jnp.full((L,), NEG_INF, jnp.float32)
    gi = jnp.zeros((L,), jnp.int32)
    for t in range(NS):
        rm = lms[2 * t]
        ri = plsc.bitcast(lms[2 * t + 1], jnp.int32)
        better = rm > gm
        gm = jnp.where(better, rm, gm)
        gi = jnp.where(better, ri, gi)

    for s in (8, 4, 2, 1):
        perm = lane ^ s
        tm[...] = gm
        ti[...] = gi
        om = plsc.load_gather(tm, [perm])
        oi = plsc.load_gather(ti, [perm])
        take = (om > gm) | ((om == gm) & (oi < gi))
        gm = jnp.where(take, om, gm)
        gi = jnp.where(take, oi, gi)

    out_dma.wait()
    wr = (gi >> 7)[0]
    owner = (wr >= wrow0) & (wr < wrow0 + W_TILE)

    @pl.when(owner)
    def _():
        off = gi & (WIDE - 1)
        rloc = wr - wrow0
        for c in range(WIDE // L):
            ov[rloc, pl.ds(c * L, L)] = jnp.where(
                lane + c * L == off, 1.0, 0.0
            ).astype(jnp.float32)
        pltpu.sync_copy(ov, out_hbm.at[pl.ds(wrow0, W_TILE), :])


def kernel(x):
    out2d = _argmax_onehot(x.reshape(ROWS, L))
    return out2d.reshape(N)
